# baseline (speedup 1.0000x reference)
import functools

import jax
import jax.numpy as jnp
from jax import lax
from jax.experimental import pallas as pl
from jax.experimental.pallas import tpu as pltpu
from jax.experimental.pallas import tpu_sc as plsc

D = 32
NC = 2
NS = 16
NW = NC * NS
CHUNK = 256
GATHER_W = 128
NBUF = 8
UNROLL = 8


def _make_kernel(E):
    per_w = E // NW
    n_chunks = per_w // CHUNK
    assert n_chunks % NBUF == 0 and n_chunks >= 2 * NBUF
    mesh = plsc.VectorSubcoreMesh(core_axis_name="c", subcore_axis_name="s")

    @functools.partial(
        pl.kernel,
        out_type=jax.ShapeDtypeStruct((E,), jnp.float32),
        mesh=mesh,
        compiler_params=pltpu.CompilerParams(
            needs_layout_passes=False, use_tc_tiling_on_sc=False),
        scratch_types=[
            [pltpu.VMEM((CHUNK,), jnp.int32)] * NBUF,
            [pltpu.VMEM((CHUNK,), jnp.int32)] * NBUF,
            [pltpu.VMEM((CHUNK, D), jnp.bfloat16)] * NBUF,
            [pltpu.VMEM((CHUNK, D), jnp.bfloat16)] * NBUF,
            [pltpu.VMEM((CHUNK,), jnp.float32)] * NBUF,
            pltpu.VMEM((CHUNK * 16,), jnp.float32),
            [pltpu.SemaphoreType.DMA] * NBUF,
            [pltpu.SemaphoreType.DMA] * NBUF,
            [pltpu.SemaphoreType.DMA] * NBUF,
        ],
    )
    def k(table, src, dst, out, idx_s, idx_d, rows_s, rows_d, scores, csum,
          sem_i, sem_g, sem_o):
        wid = lax.axis_index("s") * NC + lax.axis_index("c")
        w_base = wid * per_w

        def fire_idx(g, b):
            base = w_base + g * CHUNK
            pltpu.async_copy(src.at[pl.ds(base, CHUNK)], idx_s[b], sem_i[b])
            pltpu.async_copy(dst.at[pl.ds(base, CHUNK)], idx_d[b], sem_i[b])

        def wait_idx(b):
            pltpu.make_async_copy(
                src.at[pl.ds(0, CHUNK)], idx_s[b], sem_i[b]).wait()
            pltpu.make_async_copy(
                dst.at[pl.ds(0, CHUNK)], idx_d[b], sem_i[b]).wait()

        def fire_gathers(b):
            for j in range(CHUNK // GATHER_W):
                sl = pl.ds(j * GATHER_W, GATHER_W)
                pltpu.async_copy(
                    table.at[idx_s[b].at[sl]], rows_s[b].at[sl], sem_g[b])
                pltpu.async_copy(
                    table.at[idx_d[b].at[sl]], rows_d[b].at[sl], sem_g[b])

        def wait_gathers(b):
            for j in range(CHUNK // GATHER_W):
                sl = pl.ds(j * GATHER_W, GATHER_W)
                pltpu.make_async_copy(
                    table.at[idx_s[b].at[sl]], rows_s[b].at[sl],
                    sem_g[b]).wait()
                pltpu.make_async_copy(
                    table.at[idx_d[b].at[sl]], rows_d[b].at[sl],
                    sem_g[b]).wait()

        def fire_out(g, b):
            base = w_base + g * CHUNK
            pltpu.async_copy(scores[b], out.at[pl.ds(base, CHUNK)], sem_o[b])

        def wait_out(b):
            pltpu.make_async_copy(
                scores[b], out.at[pl.ds(0, CHUNK)], sem_o[b]).wait()

        last_lane = lax.iota(jnp.int32, 16) * 16 + 15

        def compute(b):
            def edge_body(it, c2):
                e = it * UNROLL
                for u in range(UNROLL):
                    s = rows_s[b][e + u, :]
                    t = rows_d[b][e + u, :]
                    q = s * t
                    qe, qo = plsc.unpack(q, format=plsc.PackFormat.INTERLEAVED)
                    p = qe + qo
                    csum[pl.ds((e + u) * 16, 16)] = jnp.cumsum(p)
                return c2

            lax.fori_loop(0, CHUNK // UNROLL, edge_body, 0)

            def col_body(grp, c2):
                ids = grp * 256 + last_lane
                scores[b][pl.ds(grp * 16, 16)] = plsc.load_gather(csum, [ids])
                return c2

            lax.fori_loop(0, CHUNK // 16, col_body, 0)

        for b in range(NBUF - 1):
            fire_idx(b, b)
        for b in range(NBUF - 1):
            wait_idx(b)
            fire_gathers(b)
        fire_idx(NBUF - 1, NBUF - 1)

        def ring_body(qi, carry):
            for u in range(NBUF):
                b = u
                g = qi * NBUF + u
                wait_gathers(b)
                wait_idx((u + NBUF - 1) % NBUF)
                fire_gathers((u + NBUF - 1) % NBUF)
                fire_idx(g + NBUF, b)

                @pl.when(qi > 0)
                def _():
                    wait_out(b)

                compute(b)
                fire_out(g, b)
            return carry

        lax.fori_loop(0, n_chunks // NBUF - 1, ring_body, 0)

        for u in range(NBUF):
            g = n_chunks - NBUF + u
            b = g % NBUF
            wait_gathers(b)
            if u == 0:
                wait_idx((b + NBUF - 1) % NBUF)
                fire_gathers((b + NBUF - 1) % NBUF)
            wait_out(b)
            compute(b)
            fire_out(g, b)
        for b in range(NBUF):
            wait_out(b)

    return k


def kernel(embedding, edge_index):
    E = edge_index.shape[1]
    edges = edge_index.astype(jnp.int32)
    table = embedding.astype(jnp.bfloat16)
    scores = _make_kernel(E)(table, edges[0], edges[1])
    return scores.reshape(E, 1)


# speedup vs baseline: 1.0171x; 1.0171x over previous
import functools

import jax
import jax.numpy as jnp
from jax import lax
from jax.experimental import pallas as pl
from jax.experimental.pallas import tpu as pltpu
from jax.experimental.pallas import tpu_sc as plsc

D = 32
NC = 2
NS = 16
NW = NC * NS
CHUNK = 512
GATHER_W = 128
NBUF = 4
UNROLL = 8


def _make_kernel(E):
    per_w = E // NW
    n_chunks = per_w // CHUNK
    assert n_chunks % NBUF == 0 and n_chunks >= 2 * NBUF
    mesh = plsc.VectorSubcoreMesh(core_axis_name="c", subcore_axis_name="s")

    @functools.partial(
        pl.kernel,
        out_type=jax.ShapeDtypeStruct((E,), jnp.float32),
        mesh=mesh,
        compiler_params=pltpu.CompilerParams(
            needs_layout_passes=False, use_tc_tiling_on_sc=False),
        scratch_types=[
            [pltpu.VMEM((CHUNK,), jnp.int32)] * NBUF,
            [pltpu.VMEM((CHUNK,), jnp.int32)] * NBUF,
            [pltpu.VMEM((CHUNK, D), jnp.bfloat16)] * NBUF,
            [pltpu.VMEM((CHUNK, D), jnp.bfloat16)] * NBUF,
            [pltpu.VMEM((CHUNK,), jnp.float32)] * NBUF,
            pltpu.VMEM((CHUNK * 16,), jnp.float32),
            [pltpu.SemaphoreType.DMA] * NBUF,
            [pltpu.SemaphoreType.DMA] * NBUF,
            [pltpu.SemaphoreType.DMA] * NBUF,
        ],
    )
    def k(table, src, dst, out, idx_s, idx_d, rows_s, rows_d, scores, csum,
          sem_i, sem_g, sem_o):
        wid = lax.axis_index("s") * NC + lax.axis_index("c")
        w_base = wid * per_w

        def fire_idx(g, b):
            base = w_base + g * CHUNK
            pltpu.async_copy(src.at[pl.ds(base, CHUNK)], idx_s[b], sem_i[b])
            pltpu.async_copy(dst.at[pl.ds(base, CHUNK)], idx_d[b], sem_i[b])

        def wait_idx(b):
            pltpu.make_async_copy(
                src.at[pl.ds(0, CHUNK)], idx_s[b], sem_i[b]).wait()
            pltpu.make_async_copy(
                dst.at[pl.ds(0, CHUNK)], idx_d[b], sem_i[b]).wait()

        def fire_gathers(b):
            for j in range(CHUNK // GATHER_W):
                sl = pl.ds(j * GATHER_W, GATHER_W)
                pltpu.async_copy(
                    table.at[idx_s[b].at[sl]], rows_s[b].at[sl], sem_g[b])
                pltpu.async_copy(
                    table.at[idx_d[b].at[sl]], rows_d[b].at[sl], sem_g[b])

        def wait_gathers(b):
            for j in range(CHUNK // GATHER_W):
                sl = pl.ds(j * GATHER_W, GATHER_W)
                pltpu.make_async_copy(
                    table.at[idx_s[b].at[sl]], rows_s[b].at[sl],
                    sem_g[b]).wait()
                pltpu.make_async_copy(
                    table.at[idx_d[b].at[sl]], rows_d[b].at[sl],
                    sem_g[b]).wait()

        def fire_out(g, b):
            base = w_base + g * CHUNK
            pltpu.async_copy(scores[b], out.at[pl.ds(base, CHUNK)], sem_o[b])

        def wait_out(b):
            pltpu.make_async_copy(
                scores[b], out.at[pl.ds(0, CHUNK)], sem_o[b]).wait()

        last_lane = lax.iota(jnp.int32, 16) * 16 + 15

        def compute(b):
            def edge_body(it, c2):
                e = it * UNROLL
                for u in range(UNROLL):
                    s = rows_s[b][e + u, :]
                    t = rows_d[b][e + u, :]
                    q = s * t
                    qe, qo = plsc.unpack(q, format=plsc.PackFormat.INTERLEAVED)
                    p = qe + qo
                    csum[pl.ds((e + u) * 16, 16)] = jnp.cumsum(p)
                return c2

            lax.fori_loop(0, CHUNK // UNROLL, edge_body, 0)

            def col_body(grp, c2):
                ids = grp * 256 + last_lane
                scores[b][pl.ds(grp * 16, 16)] = plsc.load_gather(csum, [ids])
                return c2

            lax.fori_loop(0, CHUNK // 16, col_body, 0)

        for b in range(NBUF - 1):
            fire_idx(b, b)
        for b in range(NBUF - 1):
            wait_idx(b)
            fire_gathers(b)
        fire_idx(NBUF - 1, NBUF - 1)

        def ring_body(qi, carry):
            for u in range(NBUF):
                b = u
                g = qi * NBUF + u
                wait_gathers(b)
                wait_idx((u + NBUF - 1) % NBUF)
                fire_gathers((u + NBUF - 1) % NBUF)
                fire_idx(g + NBUF, b)

                @pl.when(qi > 0)
                def _():
                    wait_out(b)

                compute(b)
                fire_out(g, b)
            return carry

        lax.fori_loop(0, n_chunks // NBUF - 1, ring_body, 0)

        for u in range(NBUF):
            g = n_chunks - NBUF + u
            b = g % NBUF
            wait_gathers(b)
            if u == 0:
                wait_idx((b + NBUF - 1) % NBUF)
                fire_gathers((b + NBUF - 1) % NBUF)
            wait_out(b)
            compute(b)
            fire_out(g, b)
        for b in range(NBUF):
            wait_out(b)

    return k


def kernel(embedding, edge_index):
    E = edge_index.shape[1]
    edges = edge_index.astype(jnp.int32)
    table = embedding.astype(jnp.bfloat16)
    scores = _make_kernel(E)(table, edges[0], edges[1])
    return scores.reshape(E, 1)
